# baseline (speedup 1.0000x reference)
import functools

import jax
import jax.numpy as jnp
from jax import lax
from jax.experimental import pallas as pl
from jax.experimental.pallas import tpu as pltpu
from jax.experimental.pallas import tpu_sc as plsc

_NC = 2
_NS = 16
_NW = _NC * _NS

_CHUNK = 16
_NBUF = 8

_M_BLK = 512


def _pick_chunk(rows_per_w):
    for c in range(min(_CHUNK, rows_per_w), 0, -1):
        if rows_per_w % c == 0:
            return c
    return rows_per_w


def _make_gather(num_tokens, vocab, d_model, dtype, with_dep=False):
    rows_per_w = num_tokens // _NW
    chunk = _pick_chunk(rows_per_w)
    mesh = plsc.VectorSubcoreMesh(core_axis_name="c", subcore_axis_name="s")

    @functools.partial(
        pl.kernel,
        out_type=jax.ShapeDtypeStruct((num_tokens, d_model), dtype),
        mesh=mesh,
        scratch_types=[
            pltpu.VMEM((rows_per_w,), jnp.int32),
            pltpu.VMEM((_NBUF, chunk, d_model), dtype),
        ] + [pltpu.SemaphoreType.DMA] * (2 * _NBUF),
    )
    def gather(tokens_hbm, table_hbm, *rest):
        if with_dep:
            _dep, out_hbm, idx_v, rows_v = rest[0], rest[1], rest[2], rest[3]
            sems = rest[4:]
        else:
            out_hbm, idx_v, rows_v = rest[0], rest[1], rest[2]
            sems = rest[3:]
        rsems, wsems = sems[:_NBUF], sems[_NBUF:]
        wid = lax.axis_index("s") * _NC + lax.axis_index("c")
        base = wid * rows_per_w
        n_chunks = rows_per_w // chunk

        pltpu.sync_copy(tokens_hbm.at[pl.ds(base, rows_per_w)], idx_v)

        def fire(c):
            s = c % _NBUF
            return pltpu.async_copy(
                table_hbm.at[idx_v.at[pl.ds(c * chunk, chunk)]],
                rows_v.at[s], rsems[s])

        depth = min(_NBUF - 1, n_chunks)
        rd = [None] * n_chunks
        wr = [None] * n_chunks
        for c in range(depth):
            rd[c] = fire(c)
        for c in range(n_chunks):
            rd[c].wait()
            nxt = c + depth
            if nxt < n_chunks:
                prev = nxt - _NBUF
                if prev >= 0 and wr[prev] is not None:
                    wr[prev].wait()
                    wr[prev] = None
                rd[nxt] = fire(nxt)
            wr[c] = pltpu.async_copy(
                rows_v.at[c % _NBUF],
                out_hbm.at[pl.ds(base + c * chunk, chunk)],
                wsems[c % _NBUF])
        for c in range(n_chunks):
            if wr[c] is not None:
                wr[c].wait()

    return gather


def _matmul_body(h_ref, w_ref, b_ref, o_ref):
    h = h_ref[...].astype(jnp.bfloat16)
    acc = jnp.dot(h, w_ref[...], preferred_element_type=jnp.float32)
    o_ref[...] = acc + b_ref[...]


def _matmul_alias_body(h_ref, w_ref, b_ref, _prev_ref, o_ref):
    _matmul_body(h_ref, w_ref, b_ref, o_ref)


def _make_matmul(half_tokens, num_tokens, d_model, vocab, dtype, tile_off,
                 aliased):
    m_tiles = half_tokens // _M_BLK
    in_specs = [
        pl.BlockSpec((_M_BLK, d_model), lambda i: (i, 0)),
        pl.BlockSpec((d_model, vocab), lambda i: (0, 0)),
        pl.BlockSpec((1, vocab), lambda i: (0, 0)),
    ]
    body = _matmul_body
    kwargs = {}
    if aliased:
        in_specs.append(pl.BlockSpec(memory_space=pl.ANY))
        body = _matmul_alias_body
        kwargs["input_output_aliases"] = {3: 0}
    return pl.pallas_call(
        body,
        grid=(m_tiles,),
        in_specs=in_specs,
        out_specs=pl.BlockSpec((_M_BLK, vocab),
                               lambda i: (i + tile_off, 0)),
        out_shape=jax.ShapeDtypeStruct((num_tokens, vocab), dtype),
        compiler_params=pltpu.CompilerParams(
            dimension_semantics=("arbitrary",),
        ),
        **kwargs,
    )


def kernel(tokens, emb_table, W, b):
    bsz, t = tokens.shape
    vocab, d_model = emb_table.shape
    num_tokens = bsz * t

    tok_flat = tokens.reshape(num_tokens).astype(jnp.int32)
    gather = _make_gather(num_tokens, vocab, d_model, emb_table.dtype)
    hidden = gather(tok_flat, emb_table)

    w_bf = W.astype(jnp.bfloat16)
    b2d = b.reshape(1, vocab)
    mm = _make_matmul(num_tokens, num_tokens, d_model, vocab, W.dtype, 0,
                      False)
    logits = mm(hidden, w_bf, b2d)
    return logits.reshape(bsz, t, vocab)


# speedup vs baseline: 1.0042x; 1.0042x over previous
import functools

import jax
import jax.numpy as jnp
from jax import lax
from jax.experimental import pallas as pl
from jax.experimental.pallas import tpu as pltpu
from jax.experimental.pallas import tpu_sc as plsc

_NC = 2
_NS = 16
_NW = _NC * _NS

_CHUNK = 32
_NBUF = 4

_M_BLK = 512


def _pick_chunk(rows_per_w):
    for c in range(min(_CHUNK, rows_per_w), 0, -1):
        if rows_per_w % c == 0:
            return c
    return rows_per_w


def _make_gather(num_tokens, vocab, d_model, dtype):
    rows_per_w = num_tokens // _NW
    chunk = _pick_chunk(rows_per_w)
    mesh = plsc.VectorSubcoreMesh(core_axis_name="c", subcore_axis_name="s")

    @functools.partial(
        pl.kernel,
        out_type=jax.ShapeDtypeStruct((num_tokens, d_model), dtype),
        mesh=mesh,
        scratch_types=[
            pltpu.VMEM((rows_per_w,), jnp.int32),
            pltpu.VMEM((_NBUF, chunk, d_model), dtype),
        ] + [pltpu.SemaphoreType.DMA] * (2 * _NBUF),
    )
    def gather(tokens_hbm, table_hbm, out_hbm, idx_v, rows_v, *sems):
        rsems, wsems = sems[:_NBUF], sems[_NBUF:]
        wid = lax.axis_index("s") * _NC + lax.axis_index("c")
        base = wid * rows_per_w
        n_chunks = rows_per_w // chunk

        pltpu.sync_copy(tokens_hbm.at[pl.ds(base, rows_per_w)], idx_v)

        def fire(c):
            s = c % _NBUF
            return pltpu.async_copy(
                table_hbm.at[idx_v.at[pl.ds(c * chunk, chunk)]],
                rows_v.at[s], rsems[s])

        depth = min(_NBUF - 1, n_chunks)
        rd = [None] * n_chunks
        wr = [None] * n_chunks
        for c in range(depth):
            rd[c] = fire(c)
        for c in range(n_chunks):
            rd[c].wait()
            nxt = c + depth
            if nxt < n_chunks:
                prev = nxt - _NBUF
                if prev >= 0 and wr[prev] is not None:
                    wr[prev].wait()
                    wr[prev] = None
                rd[nxt] = fire(nxt)
            wr[c] = pltpu.async_copy(
                rows_v.at[c % _NBUF],
                out_hbm.at[pl.ds(base + c * chunk, chunk)],
                wsems[c % _NBUF])
        for c in range(n_chunks):
            if wr[c] is not None:
                wr[c].wait()

    return gather


def _matmul_body(h_ref, w_ref, b_ref, o_ref):
    h = h_ref[...].astype(jnp.bfloat16)
    acc = jnp.dot(h, w_ref[...], preferred_element_type=jnp.float32)
    o_ref[...] = acc + b_ref[...]


def _make_matmul(num_tokens, d_model, vocab, dtype):
    m_tiles = num_tokens // _M_BLK
    return pl.pallas_call(
        _matmul_body,
        grid=(m_tiles,),
        in_specs=[
            pl.BlockSpec((_M_BLK, d_model), lambda i: (i, 0)),
            pl.BlockSpec((d_model, vocab), lambda i: (0, 0)),
            pl.BlockSpec((1, vocab), lambda i: (0, 0)),
        ],
        out_specs=pl.BlockSpec((_M_BLK, vocab), lambda i: (i, 0)),
        out_shape=jax.ShapeDtypeStruct((num_tokens, vocab), dtype),
        compiler_params=pltpu.CompilerParams(
            dimension_semantics=("arbitrary",),
        ),
    )


def kernel(tokens, emb_table, W, b):
    bsz, t = tokens.shape
    vocab, d_model = emb_table.shape
    num_tokens = bsz * t

    tok_flat = tokens.reshape(num_tokens).astype(jnp.int32)
    gather = _make_gather(num_tokens, vocab, d_model, emb_table.dtype)
    hidden = gather(tok_flat, emb_table)

    w_bf = W.astype(jnp.bfloat16)
    b2d = b.reshape(1, vocab)
    mm = _make_matmul(num_tokens, d_model, vocab, W.dtype)
    logits = mm(hidden, w_bf, b2d)
    return logits.reshape(bsz, t, vocab)


# speedup vs baseline: 1.0045x; 1.0003x over previous
import functools

import jax
import jax.numpy as jnp
from jax import lax
from jax.experimental import pallas as pl
from jax.experimental.pallas import tpu as pltpu
from jax.experimental.pallas import tpu_sc as plsc

_NC = 2
_NS = 16
_NW = _NC * _NS

_CHUNK = 32
_NBUF = 4

_M_BLK = 512


def _pick_chunk(rows_per_w):
    for c in range(min(_CHUNK, rows_per_w), 0, -1):
        if rows_per_w % c == 0:
            return c
    return rows_per_w


def _make_gather(num_tokens, vocab, d_model, dtype):
    rows_per_w = num_tokens // _NW
    chunk = _pick_chunk(rows_per_w)
    mesh = plsc.VectorSubcoreMesh(core_axis_name="c", subcore_axis_name="s")

    @functools.partial(
        pl.kernel,
        out_type=jax.ShapeDtypeStruct((num_tokens, d_model), dtype),
        mesh=mesh,
        scratch_types=[
            pltpu.VMEM((rows_per_w,), jnp.int32),
            pltpu.VMEM((_NBUF, chunk, d_model), dtype),
        ] + [pltpu.SemaphoreType.DMA] * (2 * _NBUF),
    )
    def gather(tokens_hbm, table_hbm, out_hbm, idx_v, rows_v, *sems):
        rsems, wsems = sems[:_NBUF], sems[_NBUF:]
        wid = lax.axis_index("s") * _NC + lax.axis_index("c")
        base = wid * rows_per_w
        n_chunks = rows_per_w // chunk

        pltpu.sync_copy(tokens_hbm.at[pl.ds(base, rows_per_w)], idx_v)

        def fire(c):
            s = c % _NBUF
            return pltpu.async_copy(
                table_hbm.at[idx_v.at[pl.ds(c * chunk, chunk)]],
                rows_v.at[s], rsems[s])

        depth = min(_NBUF - 1, n_chunks)
        rd = [None] * n_chunks
        wr = [None] * n_chunks
        for c in range(depth):
            rd[c] = fire(c)
        for c in range(n_chunks):
            rd[c].wait()
            nxt = c + depth
            if nxt < n_chunks:
                prev = nxt - _NBUF
                if prev >= 0 and wr[prev] is not None:
                    wr[prev].wait()
                    wr[prev] = None
                rd[nxt] = fire(nxt)
            wr[c] = pltpu.async_copy(
                rows_v.at[c % _NBUF],
                out_hbm.at[pl.ds(base + c * chunk, chunk)],
                wsems[c % _NBUF])
        for c in range(n_chunks):
            if wr[c] is not None:
                wr[c].wait()

    return gather


def _matmul_body(h_ref, w_ref, b_ref, o_ref):
    h = h_ref[...].astype(jnp.bfloat16)
    acc = jnp.dot(h, w_ref[...], preferred_element_type=jnp.float32)
    o_ref[...] = acc + b_ref[...]


def _make_matmul(num_tokens, d_model, vocab, dtype):
    m_tiles = num_tokens // _M_BLK
    return pl.pallas_call(
        _matmul_body,
        grid=(m_tiles,),
        in_specs=[
            pl.BlockSpec((_M_BLK, d_model), lambda i: (i, 0)),
            pl.BlockSpec((d_model, vocab), lambda i: (0, 0)),
            pl.BlockSpec((1, vocab), lambda i: (0, 0)),
        ],
        out_specs=pl.BlockSpec((_M_BLK, vocab), lambda i: (i, 0)),
        out_shape=jax.ShapeDtypeStruct((num_tokens, vocab), dtype),
        compiler_params=pltpu.CompilerParams(
            dimension_semantics=("parallel",),
        ),
    )


def kernel(tokens, emb_table, W, b):
    bsz, t = tokens.shape
    vocab, d_model = emb_table.shape
    num_tokens = bsz * t

    tok_flat = tokens.reshape(num_tokens).astype(jnp.int32)
    gather = _make_gather(num_tokens, vocab, d_model, emb_table.dtype)
    hidden = gather(tok_flat, emb_table)

    w_bf = W.astype(jnp.bfloat16)
    b2d = b.reshape(1, vocab)
    mm = _make_matmul(num_tokens, d_model, vocab, W.dtype)
    logits = mm(hidden, w_bf, b2d)
    return logits.reshape(bsz, t, vocab)
